# baseline (speedup 1.0000x reference)
import functools

import jax
import jax.numpy as jnp
from jax import lax
from jax.experimental import pallas as pl
from jax.experimental.pallas import tpu as pltpu

C1 = 384
C2 = 192
HH = 28


def _mean_body(x_ref, out_ref, *, nj, inv_hw):
    bi = pl.program_id(0)
    j = pl.program_id(1)
    partial = jnp.sum(x_ref[...], axis=(0, 1, 2))

    @pl.when(j == 0)
    def _():
        out_ref[bi, :] = jnp.zeros_like(partial)

    out_ref[bi, :] += partial

    @pl.when(j == nj - 1)
    def _():
        out_ref[bi, :] = out_ref[bi, :] * inv_hw


def _select_body(m_ref, w1_ref, w2_ref, s_ref):
    m = m_ref[...]
    y1 = lax.dot_general(m, w1_ref[...], (((1,), (1,)), ((), ())),
                         preferred_element_type=jnp.float32)
    y1 = jnp.maximum(y1, 0.0)
    z = lax.dot_general(y1, w2_ref[...], (((1,), (1,)), ((), ())),
                        preferred_element_type=jnp.float32)
    z = 1.0 / (1.0 + jnp.exp(-z))
    b = z.shape[0]
    ii = lax.broadcasted_iota(jnp.int32, (b, C1, C1), 1)
    jj = lax.broadcasted_iota(jnp.int32, (b, C1, C1), 2)
    zi = z[:, :, None]
    zj = z[:, None, :]
    before = (zj > zi) | ((zj == zi) & (jj < ii))
    rank = jnp.sum(before.astype(jnp.int32), axis=2)
    rr = lax.broadcasted_iota(jnp.int32, (b, C1, C2), 2)
    s_ref[...] = (rank[:, :, None] == rr).astype(jnp.float32)


def _gather_body(x_ref, s_ref, o_ref):
    xin = x_ref[0].reshape(-1, C1)
    o_ref[0] = lax.dot_general(
        xin, s_ref[0], (((1,), (0,)), ((), ())),
        preferred_element_type=jnp.float32).reshape(o_ref.shape[1:])


@jax.jit
def kernel(x, W1, W2):
    b, c, h, w = x.shape
    hw = h * w
    xt = jnp.transpose(x, (0, 2, 3, 1))
    nj = h // HH

    means = pl.pallas_call(
        functools.partial(_mean_body, nj=nj, inv_hw=1.0 / hw),
        grid=(b, nj),
        in_specs=[pl.BlockSpec((1, HH, w, c), lambda bi, j: (bi, j, 0, 0))],
        out_specs=pl.BlockSpec((b, c), lambda bi, j: (0, 0)),
        out_shape=jax.ShapeDtypeStruct((b, c), jnp.float32),
    )(xt)

    sel = pl.pallas_call(
        _select_body,
        out_shape=jax.ShapeDtypeStruct((b, C1, C2), jnp.float32),
    )(means, W1, W2)

    out_t = pl.pallas_call(
        _gather_body,
        grid=(b, nj),
        in_specs=[pl.BlockSpec((1, HH, w, c), lambda bi, j: (bi, j, 0, 0)),
                  pl.BlockSpec((1, C1, C2), lambda bi, j: (bi, 0, 0))],
        out_specs=pl.BlockSpec((1, HH, w, C2), lambda bi, j: (bi, j, 0, 0)),
        out_shape=jax.ShapeDtypeStruct((b, h, w, C2), jnp.float32),
    )(xt, sel)
    return jnp.transpose(out_t, (0, 3, 1, 2))


# speedup vs baseline: 2.0352x; 2.0352x over previous
import functools

import jax
import jax.numpy as jnp
from jax import lax
from jax.experimental import pallas as pl
from jax.experimental.pallas import tpu as pltpu

C1 = 384
C2 = 192
HM = 28
HG = 16


def _mean_body(x_ref, out_ref, *, nj):
    bi = pl.program_id(0)
    j = pl.program_id(1)
    v = x_ref[0].reshape(HM, HM, 8, C1)
    s = jnp.sum(jnp.sum(jnp.sum(v, axis=0), axis=0), axis=0)
    out_ref[bi * nj + j, :] = s


def _select_body(p_ref, w1_ref, w2_ref, s_ref, *, nj, inv_hw):
    p = p_ref[...].reshape(-1, nj, C1)
    a = p[:, 0:4] + p[:, 4:8]
    a = a[:, 0:2] + a[:, 2:4]
    m = (a[:, 0] + a[:, 1]) * inv_hw
    y1 = lax.dot_general(m, w1_ref[...], (((1,), (1,)), ((), ())),
                         preferred_element_type=jnp.float32)
    y1 = jnp.maximum(y1, 0.0)
    z = lax.dot_general(y1, w2_ref[...], (((1,), (1,)), ((), ())),
                        preferred_element_type=jnp.float32)
    z = 1.0 / (1.0 + jnp.exp(-z))
    b = z.shape[0]
    ii = lax.broadcasted_iota(jnp.int32, (b, C1, C1), 1)
    jj = lax.broadcasted_iota(jnp.int32, (b, C1, C1), 2)
    zi = z[:, :, None]
    zj = z[:, None, :]
    before = (zj > zi) | ((zj == zi) & (jj < ii))
    rank = jnp.sum(before.astype(jnp.int32), axis=2)
    rr = lax.broadcasted_iota(jnp.int32, (b, C1, C2), 2)
    s_ref[...] = (rank[:, :, None] == rr).astype(jnp.float32)


def _gather_body(x_ref, s_ref, o_ref):
    s = s_ref[0]
    for r in range(HG):
        xr = x_ref[0, r]
        o_ref[0, :, r, :] = lax.dot_general(
            s, xr, (((0,), (1,)), ((), ())),
            preferred_element_type=jnp.float32,
            precision=lax.Precision.HIGHEST)


@jax.jit
def kernel(x, W1, W2):
    b, c, h, w = x.shape
    hw = h * w
    xt = jnp.transpose(x, (0, 2, 3, 1))
    nj = h // HM

    psums = pl.pallas_call(
        functools.partial(_mean_body, nj=nj),
        grid=(b, nj),
        in_specs=[pl.BlockSpec((1, HM, w, c), lambda bi, j: (bi, j, 0, 0))],
        out_specs=pl.BlockSpec((b * nj, c), lambda bi, j: (0, 0)),
        out_shape=jax.ShapeDtypeStruct((b * nj, c), jnp.float32),
    )(xt)

    sel = pl.pallas_call(
        functools.partial(_select_body, nj=nj, inv_hw=1.0 / hw),
        out_shape=jax.ShapeDtypeStruct((b, C1, C2), jnp.float32),
    )(psums, W1, W2)

    out = pl.pallas_call(
        _gather_body,
        grid=(b, h // HG),
        in_specs=[pl.BlockSpec((1, HG, w, c), lambda bi, j: (bi, j, 0, 0)),
                  pl.BlockSpec((1, C1, C2), lambda bi, j: (bi, 0, 0))],
        out_specs=pl.BlockSpec((1, C2, HG, w), lambda bi, j: (bi, 0, j, 0)),
        out_shape=jax.ShapeDtypeStruct((b, C2, h, w), jnp.float32),
    )(xt, sel)
    return out
